# baseline (speedup 1.0000x reference)
import jax
import jax.numpy as jnp
from jax.experimental import pallas as pl

N = 10000
E = 160000
G = 64
POOL = 256
FC1, FC2 = 512, 256
L = 3
BN_EPS = 1e-5
EPS = 1e-7


import jax.lax


def _regressor_kernel(pooled_ref, w1_ref, b1_ref, g1_ref, bb1_ref,
                      w2_ref, b2_ref, g2_ref, bb2_ref, w3_ref, b3_ref, out_ref):
    z = jnp.dot(pooled_ref[...], w1_ref[...],
                preferred_element_type=jnp.float32) + b1_ref[...]
    z = jnp.maximum(z, 0.0)
    z = (z / jnp.sqrt(1.0 + BN_EPS)) * g1_ref[...] + bb1_ref[...]
    z = jnp.dot(z, w2_ref[...], preferred_element_type=jnp.float32) + b2_ref[...]
    z = jnp.maximum(z, 0.0)
    z = (z / jnp.sqrt(1.0 + BN_EPS)) * g2_ref[...] + bb2_ref[...]
    z = jnp.dot(z, w3_ref[...], preferred_element_type=jnp.float32) + b3_ref[...]
    out_ref[...] = z


def kernel(atom_feat, edge_attr, node_W, node_b, edge_W, edge_b, mlp_W1, mlp_b1,
           bn1_g, bn1_b, mlp_W2, mlp_b2, msg_scale, ln_g, ln_b, reg_W1, reg_b1,
           rbn1_g, rbn1_b, reg_W2, reg_b2, rbn2_g, rbn2_b, reg_W3, reg_b3,
           edge_index, batch):
    src = edge_index[0]
    dst = edge_index[1]
    readout = jnp.zeros((N, POOL), dtype=jnp.float32)
    for i in range(L):
        x = atom_feat @ node_W[i] + node_b[i]
        ea = edge_attr @ edge_W[i] + edge_b[i]
        msg = jnp.maximum(x[src] + ea, 0.0) + EPS
        seg_max = jax.ops.segment_max(msg, dst, num_segments=N)
        seg_max = jnp.where(jnp.isfinite(seg_max), seg_max, 0.0)
        ex = jnp.exp(msg - seg_max[dst])
        denom = jax.ops.segment_sum(ex, dst, num_segments=N)
        w = ex / (denom[dst] + 1e-16)
        agg = jax.ops.segment_sum(msg * w, dst, num_segments=N)
        agg_n = agg / jnp.maximum(jnp.linalg.norm(agg, axis=-1, keepdims=True), 1e-12)
        x_norm = jnp.linalg.norm(x, axis=-1, keepdims=True)
        out = agg_n * x_norm * msg_scale[i]
        out = out + x
        h = out @ mlp_W1[i] + mlp_b1[i]
        h = (h / jnp.sqrt(1.0 + BN_EPS)) * bn1_g[i] + bn1_b[i]
        h = jnp.maximum(h, 0.0)
        h = h @ mlp_W2[i] + mlp_b2[i]
        mu = jnp.mean(h, axis=-1, keepdims=True)
        var = jnp.var(h, axis=-1, keepdims=True)
        h = (h - mu) / jnp.sqrt(var + 1e-5) * ln_g[i] + ln_b[i]
        h = jnp.maximum(h, 0.0)
        readout = readout + jax.nn.softmax(h, axis=-1)
    pooled = jax.ops.segment_sum(readout, batch, num_segments=G)
    z = pl.pallas_call(
        _regressor_kernel,
        out_shape=jax.ShapeDtypeStruct((G, 1), jnp.float32),
    )(pooled, reg_W1, reg_b1.reshape(1, FC1), rbn1_g.reshape(1, FC1),
      rbn1_b.reshape(1, FC1), reg_W2, reg_b2.reshape(1, FC2),
      rbn2_g.reshape(1, FC2), rbn2_b.reshape(1, FC2), reg_W3,
      reg_b3.reshape(1, 1))
    return z


# speedup vs baseline: 1.9793x; 1.9793x over previous
import jax
import jax.lax as lax
import jax.numpy as jnp
from jax.experimental import pallas as pl
from jax.experimental.pallas import tpu as pltpu
from jax.experimental.pallas import tpu_sc as plsc

N = 10000
E = 160000
D_NODE = 256
D_EDGE = 16
H = 256
POOL = 256
FC1, FC2 = 512, 256
L = 3
G = 64
BN_EPS = 1e-5
EPS = 1e-7

NG = 2
GW = H // NG
BN = 400
NBN = N // BN
BE = 2000
NBE = E // BE
CH = 80
TPS = E // 16
NCH = TPS // CH
RPT = 624


def _encode_kernel(a_ref, w_ref, b_ref, out_ref):
    x = jnp.dot(a_ref[...], w_ref[0], preferred_element_type=jnp.float32)
    x = x + b_ref[0]
    for g in range(NG):
        out_ref[0, g] = x[:, g * GW:(g + 1) * GW]


def _encode(feat, W, b, rows, blk):
    nblk = rows // blk
    return pl.pallas_call(
        _encode_kernel,
        grid=(L, nblk),
        in_specs=[
            pl.BlockSpec((blk, feat.shape[1]), lambda l, bb: (bb, 0)),
            pl.BlockSpec((1, W.shape[1], H), lambda l, bb: (l, 0, 0)),
            pl.BlockSpec((1, 1, H), lambda l, bb: (l, 0, 0)),
        ],
        out_specs=pl.BlockSpec((1, NG, blk, GW), lambda l, bb: (l, 0, bb, 0)),
        out_shape=jax.ShapeDtypeStruct((L, NG, rows, GW), jnp.float32),
    )(feat, W, b.reshape(L, 1, H))


def _sc_edge_body(xg, eag, src_i, dst_i, zrows, den, num,
                  acc, sidx, didx, xb, eb, sem):
    c = lax.axis_index("c")
    s = lax.axis_index("s")
    row0 = s * RPT
    tail0 = 16 * RPT
    tail = N - 16 * RPT
    pltpu.sync_copy(src_i.at[s], sidx)
    pltpu.sync_copy(dst_i.at[s], didx)
    for l in range(L):
        u = l * NG + c
        for p in range(2):
            out = den if p == 0 else num
            pltpu.sync_copy(zrows.at[pl.ds(0, RPT)], acc.at[pl.ds(row0, RPT)])

            @pl.when(s == 15)
            def _zero_tail():
                pltpu.sync_copy(zrows.at[pl.ds(0, tail)],
                                acc.at[pl.ds(tail0, tail)])

            plsc.subcore_barrier()

            def chunk(k, carry):
                base = s * TPS + k * CH
                pltpu.async_copy(
                    xg.at[u].at[sidx.at[pl.ds(k * CH, CH)]], xb, sem).wait()
                pltpu.sync_copy(eag.at[u, pl.ds(base, CH)], eb)

                def row(r, c2):
                    for j in range(GW // 16):
                        sl = pl.ds(j * 16, 16)
                        m = jnp.maximum(xb[r, sl] + eb[r, sl], 0.0) + EPS
                        ex = jnp.exp(m)
                        xb[r, sl] = ex if p == 0 else m * ex
                    return c2

                lax.fori_loop(0, CH, row, 0)
                pltpu.sync_copy(xb, acc.at[didx.at[k]], add=True)
                return carry

            lax.fori_loop(0, NCH, chunk, 0)
            plsc.subcore_barrier()
            pltpu.sync_copy(acc.at[pl.ds(row0, RPT)],
                            out.at[u, pl.ds(row0, RPT)])

            @pl.when(s == 15)
            def _copy_tail():
                pltpu.sync_copy(acc.at[pl.ds(tail0, tail)],
                                out.at[u, pl.ds(tail0, tail)])

            plsc.subcore_barrier()


def _sc_edge_pass(xg, eag, src, dst, zrows):
    mesh = plsc.VectorSubcoreMesh(core_axis_name="c", subcore_axis_name="s")
    fn = pl.kernel(
        _sc_edge_body,
        out_type=(
            jax.ShapeDtypeStruct((L * NG, N, GW), jnp.float32),
            jax.ShapeDtypeStruct((L * NG, N, GW), jnp.float32),
        ),
        mesh=mesh,
        scratch_types=[
            pltpu.VMEM_SHARED((N, GW), jnp.float32),
            pltpu.VMEM((TPS,), jnp.int32),
            pltpu.VMEM((NCH, CH), jnp.int32),
            pltpu.VMEM((CH, GW), jnp.float32),
            pltpu.VMEM((CH, GW), jnp.float32),
            pltpu.SemaphoreType.DMA,
        ],
    )
    return fn(xg, eag, src.reshape(16, TPS), dst.reshape(16, NCH, CH), zrows)


def _combine_kernel(den_ref, num_ref, x_ref, oh_ref, ms_ref, w1_ref, b1_ref,
                    g1_ref, bb1_ref, w2_ref, b2_ref, lg_ref, lb_ref, out_ref):
    b = pl.program_id(0)
    l = pl.program_id(1)

    den = jnp.concatenate([den_ref[0, g] for g in range(NG)], axis=-1)
    num = jnp.concatenate([num_ref[0, g] for g in range(NG)], axis=-1)
    x = jnp.concatenate([x_ref[0, g] for g in range(NG)], axis=-1)

    agg = num / (den + 1e-16)
    nrm = jnp.sqrt(jnp.sum(agg * agg, axis=-1, keepdims=True))
    agg_n = agg / jnp.maximum(nrm, 1e-12)
    x_norm = jnp.sqrt(jnp.sum(x * x, axis=-1, keepdims=True))
    out = agg_n * x_norm * ms_ref[0]
    out = out + x

    h = jnp.dot(out, w1_ref[0], preferred_element_type=jnp.float32) + b1_ref[0]
    h = (h / jnp.sqrt(1.0 + BN_EPS)) * g1_ref[0] + bb1_ref[0]
    h = jnp.maximum(h, 0.0)
    h = jnp.dot(h, w2_ref[0], preferred_element_type=jnp.float32) + b2_ref[0]
    mu = jnp.mean(h, axis=-1, keepdims=True)
    var = jnp.mean((h - mu) * (h - mu), axis=-1, keepdims=True)
    h = (h - mu) / jnp.sqrt(var + 1e-5) * lg_ref[0] + lb_ref[0]
    h = jnp.maximum(h, 0.0)
    m = jnp.max(h, axis=-1, keepdims=True)
    ex = jnp.exp(h - m)
    sm = ex / jnp.sum(ex, axis=-1, keepdims=True)

    @pl.when(jnp.logical_and(b == 0, l == 0))
    def _():
        out_ref[...] = jnp.zeros_like(out_ref)

    out_ref[...] += lax.dot_general(
        oh_ref[...], sm, (((0,), (0,)), ((), ())),
        preferred_element_type=jnp.float32)


def _combine(den, num, xg, oh, ms_row, mlp_W1, mlp_b1, bn1_g, bn1_b,
             mlp_W2, mlp_b2, ln_g, ln_b):
    return pl.pallas_call(
        _combine_kernel,
        grid=(NBN, L),
        in_specs=[
            pl.BlockSpec((1, NG, BN, GW), lambda b, l: (l, 0, b, 0)),
            pl.BlockSpec((1, NG, BN, GW), lambda b, l: (l, 0, b, 0)),
            pl.BlockSpec((1, NG, BN, GW), lambda b, l: (l, 0, b, 0)),
            pl.BlockSpec((BN, G), lambda b, l: (b, 0)),
            pl.BlockSpec((1, 1, POOL), lambda b, l: (l, 0, 0)),
            pl.BlockSpec((1, H, 2 * H), lambda b, l: (l, 0, 0)),
            pl.BlockSpec((1, 1, 2 * H), lambda b, l: (l, 0, 0)),
            pl.BlockSpec((1, 1, 2 * H), lambda b, l: (l, 0, 0)),
            pl.BlockSpec((1, 1, 2 * H), lambda b, l: (l, 0, 0)),
            pl.BlockSpec((1, 2 * H, POOL), lambda b, l: (l, 0, 0)),
            pl.BlockSpec((1, 1, POOL), lambda b, l: (l, 0, 0)),
            pl.BlockSpec((1, 1, POOL), lambda b, l: (l, 0, 0)),
            pl.BlockSpec((1, 1, POOL), lambda b, l: (l, 0, 0)),
        ],
        out_specs=pl.BlockSpec((G, POOL), lambda b, l: (0, 0)),
        out_shape=jax.ShapeDtypeStruct((G, POOL), jnp.float32),
    )(den, num, xg, oh, ms_row.reshape(L, 1, POOL), mlp_W1,
      mlp_b1.reshape(L, 1, 2 * H), bn1_g.reshape(L, 1, 2 * H),
      bn1_b.reshape(L, 1, 2 * H), mlp_W2, mlp_b2.reshape(L, 1, POOL),
      ln_g.reshape(L, 1, POOL), ln_b.reshape(L, 1, POOL))


def _regressor_kernel(pooled_ref, w1_ref, b1_ref, g1_ref, bb1_ref,
                      w2_ref, b2_ref, g2_ref, bb2_ref, w3_ref, b3_ref, out_ref):
    z = jnp.dot(pooled_ref[...], w1_ref[...],
                preferred_element_type=jnp.float32) + b1_ref[0]
    z = jnp.maximum(z, 0.0)
    z = (z / jnp.sqrt(1.0 + BN_EPS)) * g1_ref[0] + bb1_ref[0]
    z = jnp.dot(z, w2_ref[...], preferred_element_type=jnp.float32) + b2_ref[0]
    z = jnp.maximum(z, 0.0)
    z = (z / jnp.sqrt(1.0 + BN_EPS)) * g2_ref[...] + bb2_ref[0]
    z = jnp.dot(z, w3_ref[...], preferred_element_type=jnp.float32) + b3_ref[...]
    out_ref[...] = z


def kernel(atom_feat, edge_attr, node_W, node_b, edge_W, edge_b, mlp_W1, mlp_b1,
           bn1_g, bn1_b, mlp_W2, mlp_b2, msg_scale, ln_g, ln_b, reg_W1, reg_b1,
           rbn1_g, rbn1_b, reg_W2, reg_b2, rbn2_g, rbn2_b, reg_W3, reg_b3,
           edge_index, batch):
    src = edge_index[0]
    dst = edge_index[1]

    xg = _encode(atom_feat, node_W, node_b, N, BN)
    eag = _encode(edge_attr, edge_W, edge_b, E, BE)

    zrows = jnp.zeros((640, GW), jnp.float32)
    den, num = _sc_edge_pass(xg.reshape(L * NG, N, GW),
                             eag.reshape(L * NG, E, GW), src, dst, zrows)
    den = den.reshape(L, NG, N, GW)
    num = num.reshape(L, NG, N, GW)

    oh = (batch[:, None] == jnp.arange(G, dtype=jnp.int32)[None, :])
    oh = oh.astype(jnp.float32)
    ms_row = jnp.broadcast_to(msg_scale[:, None], (L, POOL))

    pooled = _combine(den, num, xg, oh, ms_row, mlp_W1, mlp_b1, bn1_g, bn1_b,
                      mlp_W2, mlp_b2, ln_g, ln_b)

    z = pl.pallas_call(
        _regressor_kernel,
        out_shape=jax.ShapeDtypeStruct((G, 1), jnp.float32),
    )(pooled, reg_W1, reg_b1.reshape(1, FC1), rbn1_g.reshape(1, FC1),
      rbn1_b.reshape(1, FC1), reg_W2, reg_b2.reshape(1, FC2),
      rbn2_g.reshape(1, FC2), rbn2_b.reshape(1, FC2), reg_W3,
      reg_b3.reshape(1, 1))
    return z


# speedup vs baseline: 3.2616x; 1.6479x over previous
import jax
import jax.lax as lax
import jax.numpy as jnp
from jax.experimental import pallas as pl
from jax.experimental.pallas import tpu as pltpu
from jax.experimental.pallas import tpu_sc as plsc

N = 10000
E = 160000
D_NODE = 256
D_EDGE = 16
H = 256
POOL = 256
FC1, FC2 = 512, 256
L = 3
G = 64
BN_EPS = 1e-5
EPS = 1e-7

NG = 2
GW = H // NG
BN = 400
NBN = N // BN
BE = 2000
NBE = E // BE
CH = 40
TPS = E // 16
NCH = TPS // CH
RPT = 624


def _encode_kernel(a_ref, w_ref, b_ref, out_ref):
    x = jnp.dot(a_ref[...], w_ref[0], preferred_element_type=jnp.float32)
    x = x + b_ref[0]
    for g in range(NG):
        out_ref[0, g] = x[:, g * GW:(g + 1) * GW]


def _encode(feat, W, b, rows, blk):
    nblk = rows // blk
    return pl.pallas_call(
        _encode_kernel,
        grid=(L, nblk),
        in_specs=[
            pl.BlockSpec((blk, feat.shape[1]), lambda l, bb: (bb, 0)),
            pl.BlockSpec((1, W.shape[1], H), lambda l, bb: (l, 0, 0)),
            pl.BlockSpec((1, 1, H), lambda l, bb: (l, 0, 0)),
        ],
        out_specs=pl.BlockSpec((1, NG, blk, GW), lambda l, bb: (l, 0, bb, 0)),
        out_shape=jax.ShapeDtypeStruct((L, NG, rows, GW), jnp.float32),
    )(feat, W, b.reshape(L, 1, H))


def _sc_edge_body(xg, eag, src_i, dst_i, zrows, den, num,
                  acc, sidx, didx, xb0, xb1, eb0, eb1,
                  sg0, sg1, se0, se1, sd0, sd1):
    c = lax.axis_index("c")
    s = lax.axis_index("s")
    row0 = s * RPT
    tail0 = 16 * RPT
    tail = N - 16 * RPT
    xbs, ebs = (xb0, xb1), (eb0, eb1)
    sgs, ses, sds = (sg0, sg1), (se0, se1), (sd0, sd1)
    pltpu.sync_copy(src_i.at[s], sidx)

    def _gather(u, k, b):
        return pltpu.make_async_copy(
            xg.at[u].at[sidx.at[pl.ds(k * CH, CH)]], xbs[b], sgs[b])

    def _ea(u, k, b):
        return pltpu.make_async_copy(
            eag.at[u, pl.ds(s * TPS + k * CH, CH)], ebs[b], ses[b])

    def _didx(k, b):
        return pltpu.make_async_copy(dst_i.at[s, k], didx.at[b], sds[b])

    for l in range(L):
        u = l * NG + c
        for p in range(2):
            out = den if p == 0 else num
            pltpu.sync_copy(zrows.at[pl.ds(0, RPT)], acc.at[pl.ds(row0, RPT)])

            @pl.when(s == 15)
            def _zero_tail():
                pltpu.sync_copy(zrows.at[pl.ds(0, tail)],
                                acc.at[pl.ds(tail0, tail)])

            plsc.subcore_barrier()
            _gather(u, 0, 0).start()
            _ea(u, 0, 0).start()
            _didx(0, 0).start()

            def pair(k0, carry):
                for b in range(2):
                    k = k0 * 2 + b

                    @pl.when(k + 1 < NCH)
                    def _prefetch():
                        _gather(u, k + 1, 1 - b).start()
                        _ea(u, k + 1, 1 - b).start()
                        _didx(k + 1, 1 - b).start()

                    _gather(u, k, b).wait()
                    _ea(u, k, b).wait()
                    _didx(k, b).wait()

                    def row(r, c2):
                        for j in range(GW // 16):
                            sl = pl.ds(j * 16, 16)
                            m = jnp.maximum(
                                xbs[b][r, sl] + ebs[b][r, sl], 0.0) + EPS
                            ex = jnp.exp(m)
                            xbs[b][r, sl] = ex if p == 0 else m * ex
                        return c2

                    lax.fori_loop(0, CH, row, 0)
                    pltpu.sync_copy(xbs[b], acc.at[didx.at[b]], add=True)
                return carry

            lax.fori_loop(0, NCH // 2, pair, 0)
            plsc.subcore_barrier()
            pltpu.sync_copy(acc.at[pl.ds(row0, RPT)],
                            out.at[u, pl.ds(row0, RPT)])

            @pl.when(s == 15)
            def _copy_tail():
                pltpu.sync_copy(acc.at[pl.ds(tail0, tail)],
                                out.at[u, pl.ds(tail0, tail)])

            plsc.subcore_barrier()


def _sc_edge_pass(xg, eag, src, dst, zrows):
    mesh = plsc.VectorSubcoreMesh(core_axis_name="c", subcore_axis_name="s")
    fn = pl.kernel(
        _sc_edge_body,
        out_type=(
            jax.ShapeDtypeStruct((L * NG, N, GW), jnp.float32),
            jax.ShapeDtypeStruct((L * NG, N, GW), jnp.float32),
        ),
        mesh=mesh,
        scratch_types=[
            pltpu.VMEM_SHARED((N, GW), jnp.float32),
            pltpu.VMEM((TPS,), jnp.int32),
            pltpu.VMEM((2, CH), jnp.int32),
            pltpu.VMEM((CH, GW), jnp.float32),
            pltpu.VMEM((CH, GW), jnp.float32),
            pltpu.VMEM((CH, GW), jnp.float32),
            pltpu.VMEM((CH, GW), jnp.float32),
            pltpu.SemaphoreType.DMA,
            pltpu.SemaphoreType.DMA,
            pltpu.SemaphoreType.DMA,
            pltpu.SemaphoreType.DMA,
            pltpu.SemaphoreType.DMA,
            pltpu.SemaphoreType.DMA,
        ],
    )
    return fn(xg, eag, src.reshape(16, TPS), dst.reshape(16, NCH, CH), zrows)


def _combine_kernel(den_ref, num_ref, x_ref, oh_ref, ms_ref, w1_ref, b1_ref,
                    g1_ref, bb1_ref, w2_ref, b2_ref, lg_ref, lb_ref, out_ref):
    b = pl.program_id(0)
    l = pl.program_id(1)

    den = jnp.concatenate([den_ref[0, g] for g in range(NG)], axis=-1)
    num = jnp.concatenate([num_ref[0, g] for g in range(NG)], axis=-1)
    x = jnp.concatenate([x_ref[0, g] for g in range(NG)], axis=-1)

    agg = num / (den + 1e-16)
    nrm = jnp.sqrt(jnp.sum(agg * agg, axis=-1, keepdims=True))
    agg_n = agg / jnp.maximum(nrm, 1e-12)
    x_norm = jnp.sqrt(jnp.sum(x * x, axis=-1, keepdims=True))
    out = agg_n * x_norm * ms_ref[0]
    out = out + x

    h = jnp.dot(out, w1_ref[0], preferred_element_type=jnp.float32) + b1_ref[0]
    h = (h / jnp.sqrt(1.0 + BN_EPS)) * g1_ref[0] + bb1_ref[0]
    h = jnp.maximum(h, 0.0)
    h = jnp.dot(h, w2_ref[0], preferred_element_type=jnp.float32) + b2_ref[0]
    mu = jnp.mean(h, axis=-1, keepdims=True)
    var = jnp.mean((h - mu) * (h - mu), axis=-1, keepdims=True)
    h = (h - mu) / jnp.sqrt(var + 1e-5) * lg_ref[0] + lb_ref[0]
    h = jnp.maximum(h, 0.0)
    m = jnp.max(h, axis=-1, keepdims=True)
    ex = jnp.exp(h - m)
    sm = ex / jnp.sum(ex, axis=-1, keepdims=True)

    @pl.when(jnp.logical_and(b == 0, l == 0))
    def _():
        out_ref[...] = jnp.zeros_like(out_ref)

    out_ref[...] += lax.dot_general(
        oh_ref[...], sm, (((0,), (0,)), ((), ())),
        preferred_element_type=jnp.float32)


def _combine(den, num, xg, oh, ms_row, mlp_W1, mlp_b1, bn1_g, bn1_b,
             mlp_W2, mlp_b2, ln_g, ln_b):
    return pl.pallas_call(
        _combine_kernel,
        grid=(NBN, L),
        in_specs=[
            pl.BlockSpec((1, NG, BN, GW), lambda b, l: (l, 0, b, 0)),
            pl.BlockSpec((1, NG, BN, GW), lambda b, l: (l, 0, b, 0)),
            pl.BlockSpec((1, NG, BN, GW), lambda b, l: (l, 0, b, 0)),
            pl.BlockSpec((BN, G), lambda b, l: (b, 0)),
            pl.BlockSpec((1, 1, POOL), lambda b, l: (l, 0, 0)),
            pl.BlockSpec((1, H, 2 * H), lambda b, l: (l, 0, 0)),
            pl.BlockSpec((1, 1, 2 * H), lambda b, l: (l, 0, 0)),
            pl.BlockSpec((1, 1, 2 * H), lambda b, l: (l, 0, 0)),
            pl.BlockSpec((1, 1, 2 * H), lambda b, l: (l, 0, 0)),
            pl.BlockSpec((1, 2 * H, POOL), lambda b, l: (l, 0, 0)),
            pl.BlockSpec((1, 1, POOL), lambda b, l: (l, 0, 0)),
            pl.BlockSpec((1, 1, POOL), lambda b, l: (l, 0, 0)),
            pl.BlockSpec((1, 1, POOL), lambda b, l: (l, 0, 0)),
        ],
        out_specs=pl.BlockSpec((G, POOL), lambda b, l: (0, 0)),
        out_shape=jax.ShapeDtypeStruct((G, POOL), jnp.float32),
    )(den, num, xg, oh, ms_row.reshape(L, 1, POOL), mlp_W1,
      mlp_b1.reshape(L, 1, 2 * H), bn1_g.reshape(L, 1, 2 * H),
      bn1_b.reshape(L, 1, 2 * H), mlp_W2, mlp_b2.reshape(L, 1, POOL),
      ln_g.reshape(L, 1, POOL), ln_b.reshape(L, 1, POOL))


def _regressor_kernel(pooled_ref, w1_ref, b1_ref, g1_ref, bb1_ref,
                      w2_ref, b2_ref, g2_ref, bb2_ref, w3_ref, b3_ref, out_ref):
    z = jnp.dot(pooled_ref[...], w1_ref[...],
                preferred_element_type=jnp.float32) + b1_ref[0]
    z = jnp.maximum(z, 0.0)
    z = (z / jnp.sqrt(1.0 + BN_EPS)) * g1_ref[0] + bb1_ref[0]
    z = jnp.dot(z, w2_ref[...], preferred_element_type=jnp.float32) + b2_ref[0]
    z = jnp.maximum(z, 0.0)
    z = (z / jnp.sqrt(1.0 + BN_EPS)) * g2_ref[...] + bb2_ref[0]
    z = jnp.dot(z, w3_ref[...], preferred_element_type=jnp.float32) + b3_ref[...]
    out_ref[...] = z


def kernel(atom_feat, edge_attr, node_W, node_b, edge_W, edge_b, mlp_W1, mlp_b1,
           bn1_g, bn1_b, mlp_W2, mlp_b2, msg_scale, ln_g, ln_b, reg_W1, reg_b1,
           rbn1_g, rbn1_b, reg_W2, reg_b2, rbn2_g, rbn2_b, reg_W3, reg_b3,
           edge_index, batch):
    src = edge_index[0]
    dst = edge_index[1]

    xg = _encode(atom_feat, node_W, node_b, N, BN)
    eag = _encode(edge_attr, edge_W, edge_b, E, BE)

    zrows = jnp.zeros((640, GW), jnp.float32)
    den, num = _sc_edge_pass(xg.reshape(L * NG, N, GW),
                             eag.reshape(L * NG, E, GW), src, dst, zrows)
    den = den.reshape(L, NG, N, GW)
    num = num.reshape(L, NG, N, GW)

    oh = (batch[:, None] == jnp.arange(G, dtype=jnp.int32)[None, :])
    oh = oh.astype(jnp.float32)
    ms_row = jnp.broadcast_to(msg_scale[:, None], (L, POOL))

    pooled = _combine(den, num, xg, oh, ms_row, mlp_W1, mlp_b1, bn1_g, bn1_b,
                      mlp_W2, mlp_b2, ln_g, ln_b)

    z = pl.pallas_call(
        _regressor_kernel,
        out_shape=jax.ShapeDtypeStruct((G, 1), jnp.float32),
    )(pooled, reg_W1, reg_b1.reshape(1, FC1), rbn1_g.reshape(1, FC1),
      rbn1_b.reshape(1, FC1), reg_W2, reg_b2.reshape(1, FC2),
      rbn2_g.reshape(1, FC2), rbn2_b.reshape(1, FC2), reg_W3,
      reg_b3.reshape(1, 1))
    return z
